# baseline (speedup 1.0000x reference)
import jax
import jax.numpy as jnp
import numpy as np
from jax.experimental import pallas as pl
from jax.experimental.pallas import tpu as pltpu

try:
    from jax import shard_map
except ImportError:
    from jax.experimental.shard_map import shard_map

_TRUNC_R = 0.85
_NEG_FILL = -70.0
_INT_MIN32 = -2147483648


def _topp_mask_kernel(x_ref, o_ref):
    x = x_ref[0]
    iota0 = jax.lax.broadcasted_iota(jnp.int32, x.shape, 0)
    mx = jnp.max(x, axis=0, keepdims=True)
    all_easy = jnp.all(mx >= -0.16)

    @pl.when(all_easy)
    def _fast():
        first_max = jnp.min(jnp.where(x >= mx, iota0, x.shape[0]), axis=0,
                            keepdims=True)
        o_ref[0] = jnp.where(iota0 == first_max, x, _NEG_FILL)

    @pl.when(jnp.logical_not(all_easy))
    def _general():
        _topp_mask_general(x, iota0, o_ref)


def _topp_mask_general(x, iota, o_ref):
    e = jnp.exp(x)
    bits = jax.lax.bitcast_convert_type(x, jnp.int32)
    key = jnp.where(bits >= 0, bits, bits ^ 0x7FFFFFFF)
    cols = x.shape[1]

    def body(i, ans_u):
        bit = jax.lax.shift_left(jnp.int32(1), jnp.int32(31) - i)
        cand_u = ans_u | bit
        cand_s = cand_u ^ _INT_MIN32
        g = jnp.sum(jnp.where(key >= cand_s, e, 0.0), axis=0, keepdims=True)
        return jnp.where(g >= _TRUNC_R, cand_u, ans_u)

    ans_u = jax.lax.fori_loop(0, 32, body, jnp.zeros((1, cols), jnp.int32))
    t_s = ans_u ^ _INT_MIN32
    gt = key > t_s
    F = jnp.sum(jnp.where(gt, e, 0.0), axis=0, keepdims=True)
    eq = key == t_s
    t_bits = jnp.where(t_s >= 0, t_s, t_s ^ 0x7FFFFFFF)
    e_t = jnp.exp(jax.lax.bitcast_convert_type(t_bits, jnp.float32))

    def tie_body(i, a):
        bit = jax.lax.shift_left(jnp.int32(1), jnp.int32(12) - i)
        cand = a | bit
        cnt = jnp.sum(jnp.where(eq & (iota < cand), 1.0, 0.0), axis=0,
                      keepdims=True)
        ok = F + jnp.maximum(cnt - 1.0, 0.0) * e_t < _TRUNC_R
        return jnp.where(ok, cand, a)

    idx_cut = jax.lax.fori_loop(0, 13, tie_body, jnp.zeros((1, cols), jnp.int32))
    keep = gt | (eq & (iota < idx_cut))
    o_ref[0] = jnp.where(keep, x, _NEG_FILL)


def _topp_call(logits):
    b, k, s = logits.shape
    chunk = 256
    return pl.pallas_call(
        _topp_mask_kernel,
        grid=(b, s // chunk),
        in_specs=[pl.BlockSpec((1, k, chunk), lambda i, j: (i, 0, j))],
        out_specs=pl.BlockSpec((1, k, chunk), lambda i, j: (i, 0, j)),
        out_shape=jax.ShapeDtypeStruct(logits.shape, logits.dtype),
        compiler_params=pltpu.CompilerParams(
            dimension_semantics=("parallel", "parallel")),
    )(logits)


def kernel(logits):
    devs = jax.devices()
    n = next((d for d in (8, 4, 2) if d <= len(devs)
              and logits.shape[0] % d == 0), 1)
    if n == 1:
        return _topp_call(logits)
    mesh = jax.sharding.Mesh(np.array(devs[:n]), ("d",))
    pspec = jax.sharding.PartitionSpec("d", None, None)
    return shard_map(_topp_call, mesh=mesh, in_specs=pspec,
                     out_specs=pspec, check_vma=False)(logits)


# speedup vs baseline: 10.2247x; 10.2247x over previous
import jax
import jax.numpy as jnp
from jax.experimental import pallas as pl
from jax.experimental.pallas import tpu as pltpu

_TRUNC_R = 0.85
_NEG_FILL = -70.0
_INT_MIN32 = -2147483648


def _topp_mask_kernel(x_ref, o_ref):
    x = x_ref[0]
    iota0 = jax.lax.broadcasted_iota(jnp.int32, x.shape, 0)
    mx = jnp.max(x, axis=0, keepdims=True)
    all_easy = jnp.all(mx >= -0.16)

    @pl.when(all_easy)
    def _fast():
        first_max = jnp.min(jnp.where(x >= mx, iota0, x.shape[0]), axis=0,
                            keepdims=True)
        o_ref[0] = jnp.where(iota0 == first_max, x, _NEG_FILL)

    @pl.when(jnp.logical_not(all_easy))
    def _general():
        _topp_mask_general(x, iota0, o_ref)


def _topp_mask_general(x, iota, o_ref):
    e = jnp.exp(x)
    bits = jax.lax.bitcast_convert_type(x, jnp.int32)
    key = jnp.where(bits >= 0, bits, bits ^ 0x7FFFFFFF)
    cols = x.shape[1]

    def body(i, ans_u):
        bit = jax.lax.shift_left(jnp.int32(1), jnp.int32(31) - i)
        cand_u = ans_u | bit
        cand_s = cand_u ^ _INT_MIN32
        g = jnp.sum(jnp.where(key >= cand_s, e, 0.0), axis=0, keepdims=True)
        return jnp.where(g >= _TRUNC_R, cand_u, ans_u)

    ans_u = jax.lax.fori_loop(0, 32, body, jnp.zeros((1, cols), jnp.int32))
    t_s = ans_u ^ _INT_MIN32
    gt = key > t_s
    F = jnp.sum(jnp.where(gt, e, 0.0), axis=0, keepdims=True)
    eq = key == t_s
    t_bits = jnp.where(t_s >= 0, t_s, t_s ^ 0x7FFFFFFF)
    e_t = jnp.exp(jax.lax.bitcast_convert_type(t_bits, jnp.float32))

    def tie_body(i, a):
        bit = jax.lax.shift_left(jnp.int32(1), jnp.int32(12) - i)
        cand = a | bit
        cnt = jnp.sum(jnp.where(eq & (iota < cand), 1.0, 0.0), axis=0,
                      keepdims=True)
        ok = F + jnp.maximum(cnt - 1.0, 0.0) * e_t < _TRUNC_R
        return jnp.where(ok, cand, a)

    idx_cut = jax.lax.fori_loop(0, 13, tie_body, jnp.zeros((1, cols), jnp.int32))
    keep = gt | (eq & (iota < idx_cut))
    o_ref[0] = jnp.where(keep, x, _NEG_FILL)


def _topp_call(logits):
    b, k, s = logits.shape
    chunk = 512
    return pl.pallas_call(
        _topp_mask_kernel,
        grid=(b, s // chunk),
        in_specs=[pl.BlockSpec((1, k, chunk), lambda i, j: (i, 0, j))],
        out_specs=pl.BlockSpec((1, k, chunk), lambda i, j: (i, 0, j)),
        out_shape=jax.ShapeDtypeStruct(logits.shape, logits.dtype),
        compiler_params=pltpu.CompilerParams(
            dimension_semantics=("parallel", "parallel")),
    )(logits)


def kernel(logits):
    return _topp_call(logits)
